# baseline (speedup 1.0000x reference)
import functools
import math

import jax
import jax.numpy as jnp
from jax import lax
from jax.experimental import pallas as pl
from jax.experimental.pallas import tpu as pltpu
from jax.experimental.pallas import tpu_sc as plsc

_NC = 2
_NS = 16
_NW = _NC * _NS
_LANES = 16
_PAD = 64


@functools.lru_cache(maxsize=None)
def _build(nseq, seq_len, d):
    spw = nseq // _NW
    scale = jnp.float32(math.sqrt(d))
    mesh = plsc.VectorSubcoreMesh(core_axis_name="c", subcore_axis_name="s")
    in_buf = pltpu.VMEM((_PAD, d), jnp.float32)
    row_buf = pltpu.VMEM((seq_len, d), jnp.float32)

    @functools.partial(
        pl.kernel,
        mesh=mesh,
        out_type=jax.ShapeDtypeStruct((nseq, seq_len, d), jnp.float32),
        scratch_types=[
            pltpu.VMEM((spw * _PAD,), jnp.int32),
            in_buf, in_buf, row_buf, row_buf,
            pltpu.SemaphoreType.DMA,
            pltpu.SemaphoreType.DMA,
            pltpu.SemaphoreType.DMA,
            pltpu.SemaphoreType.DMA,
        ],
    )
    def emb(idx_hbm, table_hbm, out_hbm, idx_v, ib0, ib1, ob0, ob1,
            si0, si1, so0, so1):
        ib = (ib0, ib1)
        ob = (ob0, ob1)
        si = (si0, si1)
        so = (so0, so1)
        wid = lax.axis_index("s") * _NC + lax.axis_index("c")
        seq_base = wid * spw
        pltpu.sync_copy(idx_hbm.at[pl.ds(seq_base * _PAD, spw * _PAD)], idx_v)

        def gather(c):
            return pltpu.async_copy(
                table_hbm.at[idx_v.at[pl.ds(c * _PAD, _PAD)]],
                ib[c % 2], si[c % 2])

        def scale_chunk(c):
            src, dst = ib[c % 2], ob[c % 2]

            def body(i, carry):
                for j in range(d // _LANES):
                    sl = pl.ds(j * _LANES, _LANES)
                    dst[i, sl] = src[i, sl] * scale
                return carry

            lax.fori_loop(0, seq_len, body, 0)

        def put(c):
            return pltpu.async_copy(
                ob[c % 2], out_hbm.at[seq_base + c], so[c % 2])

        inc = {0: gather(0)}
        if spw > 1:
            inc[1] = gather(1)
        outc = {}
        for c in range(spw):
            inc[c].wait()
            if c >= 2:
                outc[c - 2].wait()
            scale_chunk(c)
            outc[c] = put(c)
            if c + 2 < spw:
                inc[c + 2] = gather(c + 2)
        outc[spw - 2].wait()
        outc[spw - 1].wait()

    return emb


def kernel(token_idx, table):
    nseq, seq_len = token_idx.shape
    d = table.shape[1]
    idx = jnp.pad(token_idx.astype(jnp.int32), ((0, 0), (0, _PAD - seq_len)))
    return _build(nseq, seq_len, d)(idx.reshape(-1), table)


# speedup vs baseline: 4.5631x; 4.5631x over previous
import functools
import math

import jax
import jax.numpy as jnp
from jax import lax
from jax.experimental import pallas as pl
from jax.experimental.pallas import tpu as pltpu
from jax.experimental.pallas import tpu_sc as plsc

_NC = 2
_NS = 16
_NW = _NC * _NS
_LANES = 16
_PAD = 64


@functools.lru_cache(maxsize=None)
def _build(nseq, seq_len, d):
    spw = nseq // _NW
    bulk = (seq_len // _LANES) * _LANES
    scale = jnp.float32(math.sqrt(d))
    mesh = plsc.VectorSubcoreMesh(core_axis_name="c", subcore_axis_name="s")
    row_buf = pltpu.VMEM((seq_len, d), jnp.float32)

    @functools.partial(
        pl.kernel,
        mesh=mesh,
        out_type=jax.ShapeDtypeStruct((nseq, seq_len, d), jnp.float32),
        scratch_types=[
            pltpu.VMEM((spw * _PAD,), jnp.int32),
            row_buf, row_buf, row_buf, row_buf,
            pltpu.SemaphoreType.DMA,
            pltpu.SemaphoreType.DMA,
            pltpu.SemaphoreType.DMA,
            pltpu.SemaphoreType.DMA,
        ],
    )
    def emb(idx_hbm, table_hbm, out_hbm, idx_v, ib0, ib1, ob0, ob1,
            si0, si1, so0, so1):
        ib = (ib0, ib1)
        ob = (ob0, ob1)
        si = (si0, si1)
        so = (so0, so1)
        wid = lax.axis_index("s") * _NC + lax.axis_index("c")
        seq_base = wid * spw
        pltpu.sync_copy(idx_hbm.at[pl.ds(seq_base * _PAD, spw * _PAD)], idx_v)

        def gather(c):
            cps = [pltpu.async_copy(
                table_hbm.at[idx_v.at[pl.ds(c * _PAD, bulk)]],
                ib[c % 2].at[pl.ds(0, bulk)], si[c % 2])]
            tail = idx_v[pl.ds(c * _PAD + bulk, _LANES)]
            for t in range(bulk, seq_len):
                cps.append(pltpu.async_copy(
                    table_hbm.at[tail[t - bulk]], ib[c % 2].at[t],
                    si[c % 2]))
            return cps

        def scale_chunk(c):
            src, dst = ib[c % 2], ob[c % 2]

            def body(i, carry):
                for j in range(d // _LANES):
                    sl = pl.ds(j * _LANES, _LANES)
                    dst[i, sl] = src[i, sl] * scale
                return carry

            lax.fori_loop(0, seq_len, body, 0)

        def put(c):
            return pltpu.async_copy(
                ob[c % 2], out_hbm.at[seq_base + c], so[c % 2])

        inc = {0: gather(0)}
        if spw > 1:
            inc[1] = gather(1)
        outc = {}
        for c in range(spw):
            for cp in inc[c]:
                cp.wait()
            if c >= 2:
                outc[c - 2].wait()
            scale_chunk(c)
            outc[c] = put(c)
            if c + 2 < spw:
                inc[c + 2] = gather(c + 2)
        outc[spw - 2].wait()
        outc[spw - 1].wait()

    return emb


def kernel(token_idx, table):
    nseq, seq_len = token_idx.shape
    d = table.shape[1]
    idx = jnp.pad(token_idx.astype(jnp.int32), ((0, 0), (0, _PAD - seq_len)))
    return _build(nseq, seq_len, d)(idx.reshape(-1), table)


# speedup vs baseline: 8.4304x; 1.8475x over previous
import functools
import math

import jax
import jax.numpy as jnp
from jax import lax
from jax.experimental import pallas as pl
from jax.experimental.pallas import tpu as pltpu
from jax.experimental.pallas import tpu_sc as plsc

_NC = 2
_NS = 16
_NW = _NC * _NS
_LANES = 16


@functools.lru_cache(maxsize=None)
def _build(nseq, seq_len, d):
    stripe = nseq // _NW
    bpw = stripe * seq_len
    scale = jnp.float32(math.sqrt(d))
    mesh = plsc.VectorSubcoreMesh(core_axis_name="c", subcore_axis_name="s")
    row_buf = pltpu.VMEM((stripe, d), jnp.float32)

    @functools.partial(
        pl.kernel,
        mesh=mesh,
        out_type=jax.ShapeDtypeStruct((seq_len, nseq, d), jnp.float32),
        scratch_types=[
            pltpu.VMEM((bpw,), jnp.int32),
            row_buf, row_buf, row_buf, row_buf,
            pltpu.SemaphoreType.DMA,
            pltpu.SemaphoreType.DMA,
            pltpu.SemaphoreType.DMA,
            pltpu.SemaphoreType.DMA,
        ],
    )
    def emb(idx_hbm, table_hbm, out_hbm, idx_v, ib0, ib1, ob0, ob1,
            si0, si1, so0, so1):
        ib = (ib0, ib1)
        ob = (ob0, ob1)
        si = (si0, si1)
        so = (so0, so1)
        wid = lax.axis_index("s") * _NC + lax.axis_index("c")
        batch0 = wid * stripe
        pltpu.sync_copy(idx_hbm.at[pl.ds(wid * bpw, bpw)], idx_v)

        def gather(t):
            return pltpu.async_copy(
                table_hbm.at[idx_v.at[pl.ds(t * stripe, stripe)]],
                ib[t % 2], si[t % 2])

        def scale_chunk(t):
            src, dst = ib[t % 2], ob[t % 2]

            def body(i, carry):
                for j in range(d // _LANES):
                    sl = pl.ds(j * _LANES, _LANES)
                    dst[i, sl] = src[i, sl] * scale
                return carry

            lax.fori_loop(0, stripe, body, 0)

        def put(t):
            return pltpu.async_copy(
                ob[t % 2], out_hbm.at[t, pl.ds(batch0, stripe)], so[t % 2])

        inc = {0: gather(0)}
        if seq_len > 1:
            inc[1] = gather(1)
        outc = {}
        for t in range(seq_len):
            inc[t].wait()
            if t >= 2:
                outc[t - 2].wait()
            scale_chunk(t)
            outc[t] = put(t)
            if t + 2 < seq_len:
                inc[t + 2] = gather(t + 2)
        outc[seq_len - 2].wait()
        outc[seq_len - 1].wait()

    return emb


def kernel(token_idx, table):
    nseq, seq_len = token_idx.shape
    d = table.shape[1]
    stripe = nseq // _NW
    idx_b = (token_idx.astype(jnp.int32)
             .T.reshape(seq_len, _NW, stripe)
             .transpose(1, 0, 2).reshape(-1))
    out = _build(nseq, seq_len, d)(idx_b, table)
    return out.transpose(1, 0, 2)


# speedup vs baseline: 8.4499x; 1.0023x over previous
import functools
import math

import jax
import jax.numpy as jnp
from jax import lax
from jax.experimental import pallas as pl
from jax.experimental.pallas import tpu as pltpu
from jax.experimental.pallas import tpu_sc as plsc

_NC = 2
_NS = 16
_NW = _NC * _NS
_LANES = 16


@functools.lru_cache(maxsize=None)
def _build(nseq, seq_len, d):
    stripe = nseq // _NW
    bpw = stripe * seq_len
    scale = jnp.float32(math.sqrt(d))
    mesh = plsc.VectorSubcoreMesh(core_axis_name="c", subcore_axis_name="s")
    in_buf = pltpu.VMEM((2 * stripe, d), jnp.float32)
    row_buf = pltpu.VMEM((stripe, d), jnp.float32)

    @functools.partial(
        pl.kernel,
        mesh=mesh,
        out_type=jax.ShapeDtypeStruct((seq_len, nseq, d), jnp.float32),
        scratch_types=[
            pltpu.VMEM((bpw,), jnp.int32),
            in_buf, in_buf, row_buf, row_buf,
            pltpu.SemaphoreType.DMA,
            pltpu.SemaphoreType.DMA,
            pltpu.SemaphoreType.DMA,
            pltpu.SemaphoreType.DMA,
        ],
    )
    def emb(idx_hbm, table_hbm, out_hbm, idx_v, ib0, ib1, ob0, ob1,
            si0, si1, so0, so1):
        ib = (ib0, ib1)
        ob = (ob0, ob1)
        si = (si0, si1)
        so = (so0, so1)
        wid = lax.axis_index("s") * _NC + lax.axis_index("c")
        batch0 = wid * stripe
        pltpu.sync_copy(idx_hbm.at[pl.ds(wid * bpw, bpw)], idx_v)

        ngroups = seq_len // 2

        def gather(g):
            return pltpu.async_copy(
                table_hbm.at[idx_v.at[pl.ds(g * 2 * stripe, 2 * stripe)]],
                ib[g % 2], si[g % 2])

        def scale_half(g, h):
            src = ib[g % 2].at[pl.ds(h * stripe, stripe)]
            dst = ob[h]

            def body(i, carry):
                for j in range(d // _LANES):
                    sl = pl.ds(j * _LANES, _LANES)
                    dst[i, sl] = src[i, sl] * scale
                return carry

            lax.fori_loop(0, stripe, body, 0)

        def put(g, h):
            return pltpu.async_copy(
                ob[h], out_hbm.at[2 * g + h, pl.ds(batch0, stripe)], so[h])

        inc = {0: gather(0)}
        outc = {}
        for g in range(ngroups):
            inc[g].wait()
            if g + 1 < ngroups:
                inc[g + 1] = gather(g + 1)
            for h in range(2):
                if g >= 1:
                    outc[(g - 1, h)].wait()
                scale_half(g, h)
                outc[(g, h)] = put(g, h)
        outc[(ngroups - 1, 0)].wait()
        outc[(ngroups - 1, 1)].wait()

    return emb


def kernel(token_idx, table):
    nseq, seq_len = token_idx.shape
    d = table.shape[1]
    stripe = nseq // _NW
    idx_b = (token_idx.astype(jnp.int32)
             .T.reshape(seq_len, _NW, stripe)
             .transpose(1, 0, 2).reshape(-1))
    out = _build(nseq, seq_len, d)(idx_b, table)
    return out.transpose(1, 0, 2)
